# baseline (speedup 1.0000x reference)
import functools

import jax
import jax.numpy as jnp
from jax import lax
from jax.experimental import pallas as pl
from jax.experimental.pallas import tpu as pltpu
from jax.experimental.pallas import tpu_sc as plsc

_D = 32
_L = 16


@jax.jit
def _run(champ1, champ2, table):
    B = champ1.shape[0]
    info = plsc.get_sparse_core_info()
    nw = info.num_cores * info.num_subcores
    b_per_w = B // nw

    mesh = plsc.VectorSubcoreMesh(core_axis_name="c", subcore_axis_name="s")

    @functools.partial(
        pl.kernel,
        mesh=mesh,
        compiler_params=pltpu.CompilerParams(
            needs_layout_passes=False, use_tc_tiling_on_sc=False
        ),
        out_type=jax.ShapeDtypeStruct((B,), jnp.float32),
        scratch_types=[
            pltpu.VMEM((b_per_w,), jnp.int32),
            pltpu.VMEM((b_per_w,), jnp.int32),
            pltpu.VMEM((b_per_w, _D), jnp.float32),
            pltpu.VMEM((b_per_w, _D), jnp.float32),
            pltpu.VMEM((b_per_w,), jnp.float32),
            pltpu.SemaphoreType.DMA,
            pltpu.SemaphoreType.DMA,
        ],
    )
    def k(c1_hbm, c2_hbm, table_hbm, out_hbm,
          idx1_v, idx2_v, rows1_v, rows2_v, out_v, sem1, sem2):
        wid = lax.axis_index("s") * info.num_cores + lax.axis_index("c")
        base = wid * b_per_w
        pltpu.sync_copy(c1_hbm.at[pl.ds(base, b_per_w)], idx1_v)
        pltpu.sync_copy(c2_hbm.at[pl.ds(base, b_per_w)], idx2_v)
        cp1 = pltpu.async_copy(table_hbm.at[idx1_v], rows1_v, sem1)
        cp2 = pltpu.async_copy(table_hbm.at[idx2_v], rows2_v, sem2)
        cp1.wait()
        cp2.wait()

        lane = lax.iota(jnp.int32, _L)

        def group(g, carry):
            rowidx = lane + g * _L
            acc = jnp.zeros((_L,), jnp.float32)
            for d in range(_D):
                col = jnp.full((_L,), d, jnp.int32)
                a = plsc.load_gather(rows1_v, [rowidx, col])
                b = plsc.load_gather(rows2_v, [rowidx, col])
                acc = acc + a * b
            out_v[pl.ds(g * _L, _L)] = acc
            return carry

        lax.fori_loop(0, b_per_w // _L, group, 0)
        pltpu.sync_copy(out_v, out_hbm.at[pl.ds(base, b_per_w)])

    return k(champ1.astype(jnp.int32), champ2.astype(jnp.int32), table)


def kernel(champ1, champ2, table):
    return _run(champ1, champ2, table).reshape(-1, 1, 1)
